# baseline (speedup 1.0000x reference)
import functools
import math

import jax
import jax.numpy as jnp
from jax import lax
from jax.experimental import pallas as pl
from jax.experimental.pallas import tpu as pltpu
from jax.experimental.pallas import tpu_sc as plsc

B = 1024
V = 100000
SCALE = 64.0
COS_M2 = math.cos(0.5)
SIN_M2 = math.sin(0.5)
NEG = -1e30

C_SC = 32768
CTC = V - C_SC

CBLK = 4096
NBLK = (CTC + CBLK - 1) // CBLK

NC = 2
NS = 16
NW = NC * NS
RW = B // NW
LANES = 16


def _tc_kernel(x_ref, m_ref, s_ref):
    pid = pl.program_id(0)

    @pl.when(pid == 0)
    def _init():
        m_ref[:, :] = jnp.full((B, 1), NEG, jnp.float32)
        s_ref[:, :] = jnp.zeros((B, 1), jnp.float32)

    x = x_ref[:, :]
    lanes = lax.broadcasted_iota(jnp.int32, (B, CBLK), 1)
    limit = CTC - pid * CBLK
    xm = jnp.where(lanes < limit, x, NEG)

    m_old = m_ref[:, :]
    bm = jnp.max(xm, axis=1, keepdims=True)
    m_new = jnp.maximum(m_old, bm)
    e = jnp.exp(xm - m_new)
    s_ref[:, :] = s_ref[:, :] * jnp.exp(m_old - m_new) + jnp.sum(
        e, axis=1, keepdims=True
    )
    m_ref[:, :] = m_new


def _sc_kernel(flat_ref, tgt_ref, m_out, s_out, zy_out,
               row_v, tgt_v, idx_v, zy_v, m_v, s_v, dsem, gsem):
    wid = lax.axis_index("s") * NC + lax.axis_index("c")
    r0 = wid * RW

    pltpu.sync_copy(tgt_ref.at[pl.ds(r0, RW)], tgt_v)
    for h in range(RW // LANES):
        t16 = tgt_v[pl.ds(h * LANES, LANES)]
        rows16 = lax.iota(jnp.int32, LANES) + (r0 + h * LANES)
        idx_v[pl.ds(h * LANES, LANES)] = rows16 * V + t16
    pltpu.async_copy(flat_ref.at[idx_v], zy_v, gsem).wait()

    nvec = C_SC // LANES
    unroll = 16
    outer = nvec // unroll

    def row_body(ri, _):
        r = r0 + ri
        pltpu.async_copy(
            flat_ref.at[pl.ds(r * V + CTC, C_SC)], row_v, dsem
        ).wait()

        def max_body(j, accs):
            a0, a1, a2, a3 = accs
            base = j * (unroll * LANES)
            for k in range(unroll):
                v = row_v[pl.ds(base + k * LANES, LANES)]
                if k % 4 == 0:
                    a0 = jnp.maximum(a0, v)
                elif k % 4 == 1:
                    a1 = jnp.maximum(a1, v)
                elif k % 4 == 2:
                    a2 = jnp.maximum(a2, v)
                else:
                    a3 = jnp.maximum(a3, v)
            return a0, a1, a2, a3

        neg = jnp.full((LANES,), NEG, jnp.float32)
        a0, a1, a2, a3 = lax.fori_loop(
            0, outer, max_body, (neg, neg, neg, neg)
        )
        mb = jnp.maximum(jnp.maximum(a0, a1), jnp.maximum(a2, a3))

        def sum_body(j, accs):
            s0, s1, s2, s3 = accs
            base = j * (unroll * LANES)
            for k in range(unroll):
                e = jnp.exp(row_v[pl.ds(base + k * LANES, LANES)] - mb)
                if k % 4 == 0:
                    s0 = s0 + e
                elif k % 4 == 1:
                    s1 = s1 + e
                elif k % 4 == 2:
                    s2 = s2 + e
                else:
                    s3 = s3 + e
            return s0, s1, s2, s3

        zero = jnp.zeros((LANES,), jnp.float32)
        s0, s1, s2, s3 = lax.fori_loop(
            0, outer, sum_body, (zero, zero, zero, zero)
        )
        s16 = (s0 + s1) + (s2 + s3)

        m_v[pl.ds(ri * LANES, LANES)] = mb
        s_v[pl.ds(ri * LANES, LANES)] = s16
        return 0

    lax.fori_loop(0, RW, row_body, 0)

    pltpu.sync_copy(m_v, m_out.at[pl.ds(r0 * LANES, RW * LANES)])
    pltpu.sync_copy(s_v, s_out.at[pl.ds(r0 * LANES, RW * LANES)])
    pltpu.sync_copy(zy_v, zy_out.at[pl.ds(r0, RW)])


def _combine_kernel(mtc_ref, stc_ref, msc_ref, ssc_ref, zy_ref, out_ref):
    m_tc = mtc_ref[:, :]
    s_tc = stc_ref[:, :]
    m16 = msc_ref[:, :]
    s16 = ssc_ref[:, :]
    zy = zy_ref[:, :]
    m_sc = jnp.max(m16, axis=1, keepdims=True)
    s_sc = jnp.sum(s16 * jnp.exp(m16 - m_sc), axis=1, keepdims=True)
    m = jnp.maximum(m_tc, m_sc)
    s = s_tc * jnp.exp(m_tc - m) + s_sc * jnp.exp(m_sc - m)
    c = zy * (1.0 / SCALE)
    new_zy = SCALE * (c * COS_M2 - jnp.sqrt(1.0 - c * c) * SIN_M2)
    m2 = jnp.maximum(m, new_zy)
    inner = s * jnp.exp(m - m2) - jnp.exp(zy - m2) + jnp.exp(new_zy - m2)
    lse = m2 + jnp.log(inner)
    out_ref[:, :] = jnp.sum(lse - new_zy, keepdims=True) * (1.0 / B)


_sc_call = functools.partial(
    pl.kernel,
    out_type=(
        jax.ShapeDtypeStruct((B * LANES,), jnp.float32),
        jax.ShapeDtypeStruct((B * LANES,), jnp.float32),
        jax.ShapeDtypeStruct((B,), jnp.float32),
    ),
    mesh=plsc.VectorSubcoreMesh(core_axis_name="c", subcore_axis_name="s"),
    scratch_types=[
        pltpu.VMEM((C_SC,), jnp.float32),
        pltpu.VMEM((RW,), jnp.int32),
        pltpu.VMEM((RW,), jnp.int32),
        pltpu.VMEM((RW,), jnp.float32),
        pltpu.VMEM((RW * LANES,), jnp.float32),
        pltpu.VMEM((RW * LANES,), jnp.float32),
        pltpu.SemaphoreType.DMA,
        pltpu.SemaphoreType.DMA,
    ],
)(_sc_kernel)


def kernel(fc7, weight, nembedding, target):
    flat = fc7.reshape(-1)
    m_sc, s_sc, zy = _sc_call(flat, target.astype(jnp.int32))

    m_tc, s_tc = pl.pallas_call(
        _tc_kernel,
        grid=(NBLK,),
        in_specs=[pl.BlockSpec((B, CBLK), lambda i: (0, i))],
        out_specs=[
            pl.BlockSpec((B, 1), lambda i: (0, 0)),
            pl.BlockSpec((B, 1), lambda i: (0, 0)),
        ],
        out_shape=[
            jax.ShapeDtypeStruct((B, 1), jnp.float32),
            jax.ShapeDtypeStruct((B, 1), jnp.float32),
        ],
    )(fc7)

    out = pl.pallas_call(
        _combine_kernel,
        out_shape=jax.ShapeDtypeStruct((1, 1), jnp.float32),
    )(
        m_tc,
        s_tc,
        m_sc.reshape(B, LANES),
        s_sc.reshape(B, LANES),
        zy.reshape(B, 1),
    )
    return out[0, 0]


# speedup vs baseline: 1.8749x; 1.8749x over previous
import functools
import math

import jax
import jax.numpy as jnp
from jax import lax
from jax.experimental import pallas as pl
from jax.experimental.pallas import tpu as pltpu
from jax.experimental.pallas import tpu_sc as plsc

B = 1024
V = 100000
SCALE = 64.0
COS_M2 = math.cos(0.5)
SIN_M2 = math.sin(0.5)
NEG = -1e30

C_SC = 32768

CBLK = 4096
BLK0 = C_SC // CBLK
NBLK = (V - C_SC + CBLK - 1) // CBLK

NC = 2
NS = 16
NW = NC * NS
RW = B // NW
LANES = 16
RG = 8
CCH = 4096
NCH = C_SC // CCH


def _tc_kernel(tgt_ref, x_ref, m_ref, s_ref, zy_ref):
    pid = pl.program_id(0)

    @pl.when(pid == 0)
    def _init():
        m_ref[:, :] = jnp.full((B, 1), NEG, jnp.float32)
        s_ref[:, :] = jnp.zeros((B, 1), jnp.float32)
        zy_ref[:, :] = jnp.zeros((B, 1), jnp.float32)

    x = x_ref[:, :]
    lanes = lax.broadcasted_iota(jnp.int32, (B, CBLK), 1)
    col0 = C_SC + pid * CBLK
    valid = lanes < V - col0
    xm = jnp.where(valid, x, NEG)

    m_old = m_ref[:, :]
    bm = jnp.max(xm, axis=1, keepdims=True)
    m_new = jnp.maximum(m_old, bm)
    e = jnp.exp(xm - m_new)
    s_ref[:, :] = s_ref[:, :] * jnp.exp(m_old - m_new) + jnp.sum(
        e, axis=1, keepdims=True
    )
    m_ref[:, :] = m_new

    rel = tgt_ref[:, :] - col0
    zy_ref[:, :] = zy_ref[:, :] + jnp.sum(
        jnp.where((lanes == rel) & valid, x, 0.0), axis=1, keepdims=True
    )


def _sc_kernel(fc7_ref, tgt_ref, m_out, s_out, zy_out,
               row_v, tgt_v, zy_v, m_v, s_v, dsem):
    wid = lax.axis_index("s") * NC + lax.axis_index("c")
    r0 = pl.multiple_of(wid * RW, RW)

    pltpu.sync_copy(tgt_ref.at[pl.ds(r0 * LANES, RW * LANES)], tgt_v)

    nvec = CCH // LANES
    unroll = 16
    outer = nvec // unroll
    zero = jnp.zeros((LANES,), jnp.float32)
    neg = jnp.full((LANES,), NEG, jnp.float32)
    lane_iota = lax.iota(jnp.int32, LANES)

    for g in range(RW // RG):
        rb = pl.multiple_of(r0 + g * RG, RG)

        def chunk_body(ch, carry):
            col = pl.multiple_of(ch * CCH, CCH)
            pltpu.async_copy(
                fc7_ref.at[pl.ds(rb, RG), pl.ds(col, CCH)], row_v, dsem
            ).wait()
            out_carry = []
            for r_in in range(RG):
                m_old, s_old = carry[2 * r_in], carry[2 * r_in + 1]
                ri = g * RG + r_in
                t_b16 = tgt_v[pl.ds(ri * LANES, LANES)]

                def max_body(j, accs):
                    a0, a1, a2, a3, zacc = accs
                    base = j * (unroll * LANES)
                    cb16 = jnp.full(
                        (LANES,), ch * CCH + base, jnp.int32
                    ) + lane_iota
                    for k in range(unroll):
                        v = row_v[r_in, pl.ds(base + k * LANES, LANES)]
                        hit = (cb16 + (k * LANES)) == t_b16
                        zacc = zacc + jnp.where(hit, v, zero)
                        if k % 4 == 0:
                            a0 = jnp.maximum(a0, v)
                        elif k % 4 == 1:
                            a1 = jnp.maximum(a1, v)
                        elif k % 4 == 2:
                            a2 = jnp.maximum(a2, v)
                        else:
                            a3 = jnp.maximum(a3, v)
                    return a0, a1, a2, a3, zacc

                a0, a1, a2, a3, zacc = lax.fori_loop(
                    0, outer, max_body, (neg, neg, neg, neg, zero)
                )
                plsc.addupdate(zy_v.at[pl.ds(ri * LANES, LANES)], zacc)
                bm = jnp.maximum(jnp.maximum(a0, a1), jnp.maximum(a2, a3))
                m_new = jnp.maximum(m_old, bm)
                s_scaled = s_old * jnp.exp(m_old - m_new)

                def sum_body(j, accs):
                    s0, s1, s2, s3 = accs
                    base = j * (unroll * LANES)
                    for k in range(unroll):
                        e = jnp.exp(
                            row_v[r_in, pl.ds(base + k * LANES, LANES)]
                            - m_new
                        )
                        if k % 4 == 0:
                            s0 = s0 + e
                        elif k % 4 == 1:
                            s1 = s1 + e
                        elif k % 4 == 2:
                            s2 = s2 + e
                        else:
                            s3 = s3 + e
                    return s0, s1, s2, s3

                s0, s1, s2, s3 = lax.fori_loop(
                    0, outer, sum_body, (zero, zero, zero, zero)
                )
                s_new = s_scaled + ((s0 + s1) + (s2 + s3))
                out_carry.extend([m_new, s_new])
            return tuple(out_carry)

        init = (neg, zero) * RG
        final = lax.fori_loop(0, NCH, chunk_body, init)
        for r_in in range(RG):
            ri = g * RG + r_in
            m_v[pl.ds(ri * LANES, LANES)] = final[2 * r_in]
            s_v[pl.ds(ri * LANES, LANES)] = final[2 * r_in + 1]

    pltpu.sync_copy(m_v, m_out.at[pl.ds(r0 * LANES, RW * LANES)])
    pltpu.sync_copy(s_v, s_out.at[pl.ds(r0 * LANES, RW * LANES)])
    pltpu.sync_copy(zy_v, zy_out.at[pl.ds(r0 * LANES, RW * LANES)])


def _combine_kernel(mtc_ref, stc_ref, zytc_ref, msc_ref, ssc_ref, zysc_ref,
                    out_ref):
    m_tc = mtc_ref[:, :]
    s_tc = stc_ref[:, :]
    m16 = msc_ref[:, :]
    s16 = ssc_ref[:, :]
    zy = zytc_ref[:, :] + jnp.sum(zysc_ref[:, :], axis=1, keepdims=True)
    m_sc = jnp.max(m16, axis=1, keepdims=True)
    s_sc = jnp.sum(s16 * jnp.exp(m16 - m_sc), axis=1, keepdims=True)
    m = jnp.maximum(m_tc, m_sc)
    s = s_tc * jnp.exp(m_tc - m) + s_sc * jnp.exp(m_sc - m)
    c = zy * (1.0 / SCALE)
    new_zy = SCALE * (c * COS_M2 - jnp.sqrt(1.0 - c * c) * SIN_M2)
    m2 = jnp.maximum(m, new_zy)
    inner = s * jnp.exp(m - m2) - jnp.exp(zy - m2) + jnp.exp(new_zy - m2)
    lse = m2 + jnp.log(inner)
    out_ref[:, :] = jnp.sum(lse - new_zy, keepdims=True) * (1.0 / B)


def _zero_init_kernel(zy_v):
    zero = jnp.zeros((LANES,), jnp.float32)
    for i in range(RW):
        zy_v[pl.ds(i * LANES, LANES)] = zero


def _sc_body(fc7_ref, tgt_ref, m_out, s_out, zy_out,
             row_v, tgt_v, zy_v, m_v, s_v, dsem):
    _zero_init_kernel(zy_v)
    _sc_kernel(fc7_ref, tgt_ref, m_out, s_out, zy_out,
               row_v, tgt_v, zy_v, m_v, s_v, dsem)


_sc_call = functools.partial(
    pl.kernel,
    out_type=(
        jax.ShapeDtypeStruct((B * LANES,), jnp.float32),
        jax.ShapeDtypeStruct((B * LANES,), jnp.float32),
        jax.ShapeDtypeStruct((B * LANES,), jnp.float32),
    ),
    mesh=plsc.VectorSubcoreMesh(core_axis_name="c", subcore_axis_name="s"),
    scratch_types=[
        pltpu.VMEM((RG, CCH), jnp.float32),
        pltpu.VMEM((RW * LANES,), jnp.int32),
        pltpu.VMEM((RW * LANES,), jnp.float32),
        pltpu.VMEM((RW * LANES,), jnp.float32),
        pltpu.VMEM((RW * LANES,), jnp.float32),
        pltpu.SemaphoreType.DMA,
    ],
)(_sc_body)


def kernel(fc7, weight, nembedding, target):
    tgt = target.astype(jnp.int32)
    tgt_b = jnp.broadcast_to(tgt[:, None], (B, LANES)).reshape(-1)
    m_sc, s_sc, zy_sc = _sc_call(fc7, tgt_b)

    m_tc, s_tc, zy_tc = pl.pallas_call(
        _tc_kernel,
        grid=(NBLK,),
        in_specs=[
            pl.BlockSpec((B, 1), lambda i: (0, 0)),
            pl.BlockSpec((B, CBLK), lambda i: (0, i + BLK0)),
        ],
        out_specs=[
            pl.BlockSpec((B, 1), lambda i: (0, 0)),
            pl.BlockSpec((B, 1), lambda i: (0, 0)),
            pl.BlockSpec((B, 1), lambda i: (0, 0)),
        ],
        out_shape=[
            jax.ShapeDtypeStruct((B, 1), jnp.float32),
            jax.ShapeDtypeStruct((B, 1), jnp.float32),
            jax.ShapeDtypeStruct((B, 1), jnp.float32),
        ],
    )(tgt.reshape(B, 1), fc7)

    out = pl.pallas_call(
        _combine_kernel,
        out_shape=jax.ShapeDtypeStruct((1, 1), jnp.float32),
    )(
        m_tc,
        s_tc,
        zy_tc,
        m_sc.reshape(B, LANES),
        s_sc.reshape(B, LANES),
        zy_sc.reshape(B, LANES),
    )
    return out[0, 0]


# speedup vs baseline: 5.1029x; 2.7217x over previous
import math

import jax
import jax.numpy as jnp
from jax import lax
from jax.experimental import pallas as pl
from jax.experimental.pallas import tpu as pltpu

B = 1024
V = 100000
SCALE = 64.0
COS_M2 = math.cos(0.5)
SIN_M2 = math.sin(0.5)
NEG = -1e30

CBLK = 2048
NBLK = (V + CBLK - 1) // CBLK


def _arc_kernel(tgt_ref, x_ref, out_ref, m_ref, s_ref, zy_ref):
    pid = pl.program_id(0)

    @pl.when(pid == 0)
    def _init():
        m_ref[:, :] = jnp.full((1, B), NEG, jnp.float32)
        s_ref[:, :] = jnp.zeros((1, B), jnp.float32)
        zy_ref[:, :] = jnp.zeros((1, B), jnp.float32)

    x = x_ref[:, :]
    rows = lax.broadcasted_iota(jnp.int32, (CBLK, B), 0)
    valid = rows < V - pid * CBLK
    xm = jnp.where(valid, x, NEG)

    m_old = m_ref[:, :]
    bm = jnp.max(xm, axis=0, keepdims=True)
    m_new = jnp.maximum(m_old, bm)
    e = jnp.exp(xm - m_new)
    s_ref[:, :] = s_ref[:, :] * jnp.exp(m_old - m_new) + jnp.sum(
        e, axis=0, keepdims=True
    )
    m_ref[:, :] = m_new

    rel = tgt_ref[:, :] - pid * CBLK
    zy_ref[:, :] = zy_ref[:, :] + jnp.sum(
        jnp.where((rows == rel) & valid, x, 0.0), axis=0, keepdims=True
    )

    @pl.when(pid == NBLK - 1)
    def _fin():
        m = m_ref[:, :]
        s = s_ref[:, :]
        zy = zy_ref[:, :]
        c = zy * (1.0 / SCALE)
        new_zy = SCALE * (c * COS_M2 - jnp.sqrt(1.0 - c * c) * SIN_M2)
        m2 = jnp.maximum(m, new_zy)
        inner = s * jnp.exp(m - m2) - jnp.exp(zy - m2) + jnp.exp(new_zy - m2)
        lse = m2 + jnp.log(inner)
        out_ref[:, :] = jnp.sum(lse - new_zy, keepdims=True) * (1.0 / B)


def kernel(fc7, weight, nembedding, target):
    ft = fc7.T
    tgt2d = target.reshape(1, B).astype(jnp.int32)
    out = pl.pallas_call(
        _arc_kernel,
        grid=(NBLK,),
        in_specs=[
            pl.BlockSpec((1, B), lambda i: (0, 0)),
            pl.BlockSpec((CBLK, B), lambda i: (i, 0)),
        ],
        out_specs=pl.BlockSpec((1, 1), lambda i: (0, 0)),
        out_shape=jax.ShapeDtypeStruct((1, 1), jnp.float32),
        scratch_shapes=[
            pltpu.VMEM((1, B), jnp.float32),
            pltpu.VMEM((1, B), jnp.float32),
            pltpu.VMEM((1, B), jnp.float32),
        ],
    )(tgt2d, ft)
    return out[0, 0]


# speedup vs baseline: 6.4277x; 1.2596x over previous
import math

import jax
import jax.numpy as jnp
from jax import lax
from jax.experimental import pallas as pl
from jax.experimental.pallas import tpu as pltpu

B = 1024
V = 100000
SCALE = 64.0
COS_M2 = math.cos(0.5)
SIN_M2 = math.sin(0.5)
NEG = -1e30

CBLK = 2048
NBLK = (V + CBLK - 1) // CBLK


def _arc_kernel(tgt_ref, x_ref, out_ref, m_ref, s_ref, zy_ref):
    pid = pl.program_id(0)

    @pl.when(pid == 0)
    def _init():
        m_ref[:, :] = jnp.full((1, B), NEG, jnp.float32)
        s_ref[:, :] = jnp.zeros((1, B), jnp.float32)
        zy_ref[:, :] = jnp.zeros((1, B), jnp.float32)

    x = x_ref[:, :]
    rows = lax.broadcasted_iota(jnp.int32, (CBLK, B), 0)

    def _step(xm):
        m_old = m_ref[:, :]
        bm = jnp.max(xm, axis=0, keepdims=True)
        m_new = jnp.maximum(m_old, bm)
        e = jnp.exp(xm - m_new)
        s_ref[:, :] = s_ref[:, :] * jnp.exp(m_old - m_new) + jnp.sum(
            e, axis=0, keepdims=True
        )
        m_ref[:, :] = m_new

        rel = tgt_ref[:, :] - pid * CBLK
        zy_ref[:, :] = zy_ref[:, :] + jnp.sum(
            jnp.where(rows == rel, x, 0.0), axis=0, keepdims=True
        )

    @pl.when(pid < NBLK - 1)
    def _full():
        _step(x)

    @pl.when(pid == NBLK - 1)
    def _tail():
        _step(jnp.where(rows < V - pid * CBLK, x, NEG))

    @pl.when(pid == NBLK - 1)
    def _fin():
        m = m_ref[:, :]
        s = s_ref[:, :]
        zy = zy_ref[:, :]
        c = zy * (1.0 / SCALE)
        new_zy = SCALE * (c * COS_M2 - jnp.sqrt(1.0 - c * c) * SIN_M2)
        m2 = jnp.maximum(m, new_zy)
        inner = s * jnp.exp(m - m2) - jnp.exp(zy - m2) + jnp.exp(new_zy - m2)
        lse = m2 + jnp.log(inner)
        out_ref[:, :] = jnp.sum(lse - new_zy, keepdims=True) * (1.0 / B)


def kernel(fc7, weight, nembedding, target):
    ft = fc7.T
    tgt2d = target.reshape(1, B).astype(jnp.int32)
    out = pl.pallas_call(
        _arc_kernel,
        grid=(NBLK,),
        in_specs=[
            pl.BlockSpec((1, B), lambda i: (0, 0)),
            pl.BlockSpec((CBLK, B), lambda i: (i, 0)),
        ],
        out_specs=pl.BlockSpec((1, 1), lambda i: (0, 0)),
        out_shape=jax.ShapeDtypeStruct((1, 1), jnp.float32),
        scratch_shapes=[
            pltpu.VMEM((1, B), jnp.float32),
            pltpu.VMEM((1, B), jnp.float32),
            pltpu.VMEM((1, B), jnp.float32),
        ],
    )(tgt2d, ft)
    return out[0, 0]
